# baseline (speedup 1.0000x reference)
import functools

import jax
import jax.numpy as jnp
from jax import lax
from jax.experimental import pallas as pl
from jax.experimental.pallas import tpu as pltpu
from jax.experimental.pallas import tpu_sc as plsc

_B, _T, _K, _L = 16, 256, 8, 1024
_H, _DE, _DF = 256, 64, 32
_V = 23
_NW = 32
_TB = 256
_TW = (_B * _T) // _NW


def _sc_gather(ee_tab, ct_flat):
    mesh = plsc.VectorSubcoreMesh(core_axis_name="c", subcore_axis_name="s")

    @functools.partial(
        pl.kernel,
        out_type=jax.ShapeDtypeStruct((_B, _K, 2, _TW, 128), jnp.float32),
        mesh=mesh,
        scratch_types=[
            pltpu.VMEM((_TW,), jnp.int32),
            pltpu.VMEM((_K, _TW), jnp.int32),
            pltpu.VMEM((2, _TW, 128), jnp.float32),
            pltpu.VMEM((2, _TW, 128), jnp.float32),
            pltpu.SemaphoreType.DMA,
            pltpu.SemaphoreType.DMA,
        ],
        compiler_params=pltpu.CompilerParams(use_tc_tiling_on_sc=True),
    )
    def k(ee_hbm, ct_hbm, ee_out, c_v, eidx_v, bufa_v, bufb_v, sema, semb):
        wid = lax.axis_index("s") * 2 + lax.axis_index("c")
        b = wid // 2
        half = wid % 2
        base_t = wid * _TW
        pltpu.sync_copy(ct_hbm.at[pl.ds(base_t, _TW)], c_v)
        for g in range(_TW // 16):
            c16 = c_v[pl.ds(g * 16, 16)]
            ch = lax.shift_right_logical(c16, 1)
            for kk in range(_K):
                eidx_v[kk, pl.ds(g * 16, 16)] = ch + (b * _K + kk) * (_L // 2)
        bufs = (bufa_v, bufb_v)
        sems = (sema, semb)
        pend = [None, None]
        for r in range(_K // 2):
            i = r % 2
            if pend[i] is not None:
                pr, cps = pend[i]
                for cp in cps:
                    cp.wait()
                pltpu.sync_copy(bufs[i], ee_out.at[b, pl.ds(pr * 2, 2), half])
            pend[i] = (r, [
                pltpu.async_copy(
                    ee_hbm.at[eidx_v.at[r * 2 + j]], bufs[i].at[j], sems[i])
                for j in range(2)
            ])
        for i in (0, 1):
            pr, cps = pend[i]
            for cp in cps:
                cp.wait()
            pltpu.sync_copy(bufs[i], ee_out.at[b, pl.ds(pr * 2, 2), half])

    return k(ee_tab, ct_flat)


def _tc1_body(hs_ref, cf_ref, aa_ref, ct_ref, w1h_ref, w1f_ref, b1_ref,
              base_ref, aag_ref):
    hs = hs_ref[0]
    a = jnp.dot(hs, w1h_ref[...], preferred_element_type=jnp.float32)
    rows = lax.broadcasted_iota(jnp.int32, (_TB, _TB), 0)
    cols = lax.broadcasted_iota(jnp.int32, (_TB, _TB), 1)
    eye = (rows == cols).astype(jnp.float32)
    ctf = ct_ref[0].astype(jnp.float32)
    ct_col = lax.dot_general(
        eye, ctf, (((1,), (1,)), ((), ())),
        preferred_element_type=jnp.float32)
    ll = lax.broadcasted_iota(jnp.int32, (_TB, _L), 1).astype(jnp.float32)
    onehot = (ll == ct_col).astype(jnp.float32)
    cfg = jnp.dot(onehot, cf_ref[0],
                  preferred_element_type=jnp.float32)
    c = jnp.dot(cfg, w1f_ref[...], preferred_element_type=jnp.float32)
    base_ref[0] = a + c + b1_ref[...]
    aaf = aa_ref[0].astype(jnp.float32)
    aag_ref[0] = lax.dot_general(
        aaf, onehot, (((1,), (1,)), ((), ())),
        preferred_element_type=jnp.float32)


def _tc1_compute(hs_r, cf_t, aa_t, ct_r, w1h, w1f, b1r):
    return pl.pallas_call(
        _tc1_body,
        grid=(_B,),
        in_specs=[
            pl.BlockSpec((1, _TB, _H), lambda i: (i, 0, 0)),
            pl.BlockSpec((1, _L, _DF), lambda i: (i, 0, 0)),
            pl.BlockSpec((1, _K, _L), lambda i: (i, 0, 0)),
            pl.BlockSpec((1, 1, _TB), lambda i: (i, 0, 0)),
            pl.BlockSpec((_H, _H), lambda i: (0, 0)),
            pl.BlockSpec((_DF, _H), lambda i: (0, 0)),
            pl.BlockSpec((1, _H), lambda i: (0, 0)),
        ],
        out_specs=[
            pl.BlockSpec((1, _TB, _H), lambda i: (i, 0, 0)),
            pl.BlockSpec((1, _K, _TB), lambda i: (i, 0, 0)),
        ],
        out_shape=[
            jax.ShapeDtypeStruct((_B, _TB, _H), jnp.float32),
            jax.ShapeDtypeStruct((_B, _K, _TB), jnp.float32),
        ],
        compiler_params=pltpu.CompilerParams(
            dimension_semantics=("parallel",)),
    )(hs_r, cf_t, aa_t, ct_r, w1h, w1f, b1r)


def _tc2_body(ee_ref, ct_ref, base_ref, aag_ref, w1e_ref, w2_ref,
              p_ref, lam_ref):
    ee128 = ee_ref[0]
    odd = jnp.bitwise_and(ct_ref[0], 1)[0][None, :, None] == 1
    ee64 = jnp.where(odd, ee128[:, :, _DE:], ee128[:, :, :_DE])
    e = jnp.dot(ee64.reshape(_K * _TB, _DE), w1e_ref[...],
                preferred_element_type=jnp.float32)
    hid = jnp.maximum(e.reshape(_K, _TB, _H) + base_ref[0][None], 0.0)
    scores = jnp.sum(hid * w2_ref[...][None], axis=-1)
    m = jnp.max(scores, axis=0, keepdims=True)
    ex = jnp.exp(scores - m)
    w = ex / jnp.sum(ex, axis=0, keepdims=True)
    rows = lax.broadcasted_iota(jnp.int32, (_TB, _TB), 0)
    cols = lax.broadcasted_iota(jnp.int32, (_TB, _TB), 1)
    eye = (rows == cols).astype(jnp.float32)
    lam_ref[0] = lax.dot_general(
        eye, w, (((1,), (1,)), ((), ())),
        preferred_element_type=jnp.float32)
    aag = aag_ref[0]
    vv = lax.broadcasted_iota(jnp.int32, (_K, _TB, _V), 2).astype(jnp.float32)
    p_ref[0] = jnp.sum(
        jnp.where(aag[:, :, None] == vv, w[:, :, None], 0.0), axis=0)


def _tc2_compute(ee_r, ct_r, base, aag, w1e, w2r):
    return pl.pallas_call(
        _tc2_body,
        grid=(_B,),
        in_specs=[
            pl.BlockSpec((1, _K, _TB, 128), lambda i: (i, 0, 0, 0)),
            pl.BlockSpec((1, 1, _TB), lambda i: (i, 0, 0)),
            pl.BlockSpec((1, _TB, _H), lambda i: (i, 0, 0)),
            pl.BlockSpec((1, _K, _TB), lambda i: (i, 0, 0)),
            pl.BlockSpec((_DE, _H), lambda i: (0, 0)),
            pl.BlockSpec((1, _H), lambda i: (0, 0)),
        ],
        out_specs=[
            pl.BlockSpec((1, _TB, _V), lambda i: (i, 0, 0)),
            pl.BlockSpec((1, _TB, _K), lambda i: (i, 0, 0)),
        ],
        out_shape=[
            jax.ShapeDtypeStruct((_B, _TB, _V), jnp.float32),
            jax.ShapeDtypeStruct((_B, _TB, _K), jnp.float32),
        ],
        compiler_params=pltpu.CompilerParams(
            dimension_semantics=("parallel",)),
    )(ee_r, ct_r, base, aag, w1e, w2r)


def kernel(hidden_states, exemplar_embeddings, column_features, c_t,
           exemplar_aa_ids, W1, b1, W2, b2):
    ee_tab = exemplar_embeddings.reshape(_B * _K * _L // 2, 128)
    ct_flat = c_t.reshape(_B * _T)

    ee_g = _sc_gather(ee_tab, ct_flat)

    hs_r = hidden_states.reshape(_B, _TB, _H)
    w1h = W1[:_H]
    w1e = W1[_H:_H + _DE]
    w1f = W1[_H + _DE:]
    b1r = b1.reshape(1, _H)
    w2r = W2.reshape(1, _H)

    ct_r = ct_flat.reshape(_B, 1, _TB)
    base, aag = _tc1_compute(
        hs_r, column_features, exemplar_aa_ids, ct_r, w1h, w1f, b1r)
    p_blocks, lam_blocks = _tc2_compute(
        ee_g.reshape(_B, _K, _TB, 128), ct_r, base, aag, w1e, w2r)
    return (p_blocks.reshape(_B, _T, _V), lam_blocks.reshape(_B, _T, _K))


# speedup vs baseline: 1.0925x; 1.0925x over previous
import functools

import jax
import jax.numpy as jnp
from jax import lax
from jax.experimental import pallas as pl
from jax.experimental.pallas import tpu as pltpu
from jax.experimental.pallas import tpu_sc as plsc

_B, _T, _K, _L = 16, 256, 8, 1024
_H, _DE, _DF = 256, 64, 32
_V = 23
_NW = 32
_TB = 256
_TW = (_B * _T) // _NW


def _sc_gather(ee_tab, ct_flat):
    mesh = plsc.VectorSubcoreMesh(core_axis_name="c", subcore_axis_name="s")

    @functools.partial(
        pl.kernel,
        out_type=jax.ShapeDtypeStruct((_B, _K, 2, _TW, 128), jnp.float32),
        mesh=mesh,
        scratch_types=[
            pltpu.VMEM((_TW,), jnp.int32),
            pltpu.VMEM((_K, _TW), jnp.int32),
            pltpu.VMEM((2, _TW, 128), jnp.float32),
            pltpu.VMEM((2, _TW, 128), jnp.float32),
            pltpu.SemaphoreType.DMA,
            pltpu.SemaphoreType.DMA,
        ],
        compiler_params=pltpu.CompilerParams(use_tc_tiling_on_sc=True),
    )
    def k(ee_hbm, ct_hbm, ee_out, c_v, eidx_v, bufa_v, bufb_v, sema, semb):
        wid = lax.axis_index("s") * 2 + lax.axis_index("c")
        b = wid // 2
        half = wid % 2
        base_t = wid * _TW
        pltpu.sync_copy(ct_hbm.at[pl.ds(base_t, _TW)], c_v)
        for g in range(_TW // 16):
            c16 = c_v[pl.ds(g * 16, 16)]
            for kk in range(_K):
                eidx_v[kk, pl.ds(g * 16, 16)] = c16 + (b * _K + kk) * _L
        bufs = (bufa_v, bufb_v)
        sems = (sema, semb)
        pend = [None, None]
        for r in range(_K // 2):
            i = r % 2
            if pend[i] is not None:
                pr, cps = pend[i]
                for cp in cps:
                    cp.wait()
                pltpu.sync_copy(bufs[i], ee_out.at[b, pl.ds(pr * 2, 2), half])
            pend[i] = (r, [
                pltpu.async_copy(
                    ee_hbm.at[eidx_v.at[r * 2 + j]], bufs[i].at[j], sems[i])
                for j in range(2)
            ])
        for i in (0, 1):
            pr, cps = pend[i]
            for cp in cps:
                cp.wait()
            pltpu.sync_copy(bufs[i], ee_out.at[b, pl.ds(pr * 2, 2), half])

    return k(ee_tab, ct_flat)


def _tc_body(hs_ref, ee_ref, cf_ref, aa_ref, ct_ref, w1h_ref, w1e_ref,
             w1f_ref, b1_ref, w2_ref, p_ref, lam_ref):
    hs = hs_ref[0]
    a = jnp.dot(hs, w1h_ref[...], preferred_element_type=jnp.float32)
    rows = lax.broadcasted_iota(jnp.int32, (_TB, _TB), 0)
    cols = lax.broadcasted_iota(jnp.int32, (_TB, _TB), 1)
    eye = (rows == cols).astype(jnp.float32)
    ctf = ct_ref[0].astype(jnp.float32)
    ct_col = lax.dot_general(
        eye, ctf, (((1,), (1,)), ((), ())),
        preferred_element_type=jnp.float32)
    ll = lax.broadcasted_iota(jnp.int32, (_TB, _L), 1).astype(jnp.float32)
    onehot = (ll == ct_col).astype(jnp.float32)
    cfg = jnp.dot(onehot, cf_ref[0],
                  preferred_element_type=jnp.float32)
    c = jnp.dot(cfg, w1f_ref[...], preferred_element_type=jnp.float32)
    base = a + c + b1_ref[...]
    ee64 = ee_ref[0][:, :, :_DE]
    e = jnp.dot(ee64.reshape(_K * _TB, _DE), w1e_ref[...],
                preferred_element_type=jnp.float32)
    hid = jnp.maximum(e.reshape(_K, _TB, _H) + base[None], 0.0)
    scores = jnp.sum(hid * w2_ref[...][None], axis=-1)
    m = jnp.max(scores, axis=0, keepdims=True)
    ex = jnp.exp(scores - m)
    w = ex / jnp.sum(ex, axis=0, keepdims=True)
    lam_ref[0] = lax.dot_general(
        eye, w, (((1,), (1,)), ((), ())),
        preferred_element_type=jnp.float32)
    aaf = aa_ref[0].astype(jnp.float32)
    aag = lax.dot_general(
        aaf, onehot, (((1,), (1,)), ((), ())),
        preferred_element_type=jnp.float32)
    vv = lax.broadcasted_iota(jnp.int32, (_K, _TB, _V), 2).astype(jnp.float32)
    p_ref[0] = jnp.sum(
        jnp.where(aag[:, :, None] == vv, w[:, :, None], 0.0), axis=0)


def _tc_compute(hs_r, ee_r, cf_t, aa_t, ct_r, w1h, w1e, w1f, b1r, w2r):
    return pl.pallas_call(
        _tc_body,
        grid=(_B,),
        in_specs=[
            pl.BlockSpec((1, _TB, _H), lambda i: (i, 0, 0)),
            pl.BlockSpec((1, _K, _TB, 128), lambda i: (i, 0, 0, 0)),
            pl.BlockSpec((1, _L, _DF), lambda i: (i, 0, 0)),
            pl.BlockSpec((1, _K, _L), lambda i: (i, 0, 0)),
            pl.BlockSpec((1, 1, _TB), lambda i: (i, 0, 0)),
            pl.BlockSpec((_H, _H), lambda i: (0, 0)),
            pl.BlockSpec((_DE, _H), lambda i: (0, 0)),
            pl.BlockSpec((_DF, _H), lambda i: (0, 0)),
            pl.BlockSpec((1, _H), lambda i: (0, 0)),
            pl.BlockSpec((1, _H), lambda i: (0, 0)),
        ],
        out_specs=[
            pl.BlockSpec((1, _TB, _V), lambda i: (i, 0, 0)),
            pl.BlockSpec((1, _TB, _K), lambda i: (i, 0, 0)),
        ],
        out_shape=[
            jax.ShapeDtypeStruct((_B, _TB, _V), jnp.float32),
            jax.ShapeDtypeStruct((_B, _TB, _K), jnp.float32),
        ],
        compiler_params=pltpu.CompilerParams(
            dimension_semantics=("parallel",)),
    )(hs_r, ee_r, cf_t, aa_t, ct_r, w1h, w1e, w1f, b1r, w2r)


def kernel(hidden_states, exemplar_embeddings, column_features, c_t,
           exemplar_aa_ids, W1, b1, W2, b2):
    ee_tab = jnp.pad(exemplar_embeddings, ((0, 0), (0, 0), (0, 0), (0, 64))
                     ).reshape(_B * _K * _L, 128)
    ct_flat = c_t.reshape(_B * _T)

    ee_g = _sc_gather(ee_tab, ct_flat)

    hs_r = hidden_states.reshape(_B, _TB, _H)
    w1h = W1[:_H]
    w1e = W1[_H:_H + _DE]
    w1f = W1[_H + _DE:]
    b1r = b1.reshape(1, _H)
    w2r = W2.reshape(1, _H)

    p_blocks, lam_blocks = _tc_compute(
        hs_r, ee_g.reshape(_B, _K, _TB, 128), column_features,
        exemplar_aa_ids, ct_flat.reshape(_B, 1, _TB), w1h, w1e, w1f,
        b1r, w2r)
    return (p_blocks.reshape(_B, _T, _V), lam_blocks.reshape(_B, _T, _K))
